# baseline (speedup 1.0000x reference)
import functools

import jax
import jax.numpy as jnp
from jax import lax
from jax.experimental import pallas as pl
from jax.experimental.pallas import tpu as pltpu
from jax.experimental.pallas import tpu_sc as plsc

_LANES = 16
_BB = 128


def _make_lookup(B, F, V, D, NC, NS):
    NW = NC * NS
    n_blocks = F * (B // _BB)
    assert n_blocks % NW == 0
    blk_w = n_blocks // NW
    b_blocks = B // _BB

    mesh = plsc.VectorSubcoreMesh(core_axis_name="c", subcore_axis_name="s")

    @functools.partial(
        pl.kernel,
        mesh=mesh,
        out_type=jax.ShapeDtypeStruct((F, D, B), jnp.float32),
        scratch_types=[
            pltpu.VMEM((_BB,), jnp.int32),
            pltpu.VMEM((_BB,), jnp.int32),
            pltpu.VMEM((_BB,), jnp.int32),
            pltpu.VMEM((_BB, 2 * D), jnp.float32),
            pltpu.VMEM((D, _BB), jnp.float32),
            pltpu.SemaphoreType.DMA,
            pltpu.SemaphoreType.DMA,
            pltpu.SemaphoreType.DMA,
        ],
        compiler_params=pltpu.CompilerParams(
            use_tc_tiling_on_sc=True, needs_layout_passes=False
        ),
    )
    def lookup_kernel(
        xt_hbm, w2_hbm, out_hbm, xb_v, idx_v, off_v, g_v, gt_v, isem, gsem,
        osem,
    ):
        wid = lax.axis_index("s") * NC + lax.axis_index("c")

        def block(t, carry):
            bid = wid * blk_w + t
            f = bid // b_blocks
            b0 = (bid % b_blocks) * _BB
            pltpu.async_copy(
                xt_hbm.at[f, pl.ds(b0, _BB)], xb_v, isem
            ).wait()
            for c in range(_BB // _LANES):
                xv = xb_v[pl.ds(_LANES * c, _LANES)]
                idx_v[pl.ds(_LANES * c, _LANES)] = xv >> 1
                off_v[pl.ds(_LANES * c, _LANES)] = (xv & 1) * D
            pltpu.async_copy(w2_hbm.at[idx_v], g_v, gsem).wait()
            iota = lax.iota(jnp.int32, _LANES)
            for c in range(_BB // _LANES):
                rv = iota + (_LANES * c)
                offc = off_v[pl.ds(_LANES * c, _LANES)]
                for d in range(D):
                    gt_v[d, pl.ds(_LANES * c, _LANES)] = plsc.load_gather(
                        g_v, [rv, offc + d]
                    )
            pltpu.async_copy(
                gt_v, out_hbm.at[f, pl.ds(0, D), pl.ds(b0, _BB)], osem
            ).wait()
            return carry

        lax.fori_loop(0, blk_w, block, 0)

    return lookup_kernel


def kernel(x, weight):
    B, F = x.shape
    V, D = weight.shape
    info = plsc.get_sparse_core_info()
    w2 = weight.reshape(V // 2, 2 * D)
    out_t = _make_lookup(B, F, V, D, info.num_cores, info.num_subcores)(
        x.T, w2
    )
    return out_t.transpose(2, 0, 1)


# speedup vs baseline: 1.6201x; 1.6201x over previous
import functools

import jax
import jax.numpy as jnp
from jax import lax
from jax.experimental import pallas as pl
from jax.experimental.pallas import tpu as pltpu
from jax.experimental.pallas import tpu_sc as plsc

_LANES = 16


def _make_lookup(B, F, V, D, NC, NS):
    NW = NC * NS
    H = V // 2
    assert B % NW == 0
    rows_w = B // NW
    R = 16
    assert rows_w % R == 0
    n_ch = rows_w // R

    mesh = plsc.VectorSubcoreMesh(core_axis_name="c", subcore_axis_name="s")

    @functools.partial(
        pl.kernel,
        mesh=mesh,
        out_type=jax.ShapeDtypeStruct((B, F, D), jnp.float32),
        scratch_types=[
            pltpu.VMEM((R, F), jnp.int32),
            pltpu.VMEM((R * F, D), jnp.float32),
            pltpu.SemaphoreType.DMA,
            pltpu.SemaphoreType.DMA,
        ],
        compiler_params=pltpu.CompilerParams(use_tc_tiling_on_sc=True),
    )
    def lookup_kernel(x_hbm, w3_hbm, out_hbm, idx_v, rows_v, gsem, osem):
        wid = lax.axis_index("s") * NC + lax.axis_index("c")
        base = wid * rows_w

        def chunk(g, carry):
            r0 = base + g * R
            pltpu.sync_copy(x_hbm.at[pl.ds(r0, R)], idx_v)

            def row(r, c2):
                va = idx_v[r, pl.ds(0, _LANES)]
                vb = idx_v[r, pl.ds(F - _LANES, _LANES)]
                sv = [va[j] for j in range(_LANES)]
                sv += [vb[j] for j in range(2 * _LANES - F, _LANES)]
                for f in range(F):
                    s = sv[f]
                    a = jnp.where(s >= H, 1, 0)
                    p = s - a * H
                    pltpu.async_copy(
                        w3_hbm.at[a, pl.ds(p, 1)],
                        rows_v.at[pl.ds(r * F + f, 1)],
                        gsem,
                    )
                for f in range(F):
                    pltpu.make_async_copy(
                        w3_hbm.at[0, pl.ds(0, 1)],
                        rows_v.at[pl.ds(r * F + f, 1)],
                        gsem,
                    ).wait()
                pltpu.async_copy(
                    rows_v.at[pl.ds(r * F, F)], out_hbm.at[r0 + r], osem
                )
                return c2

            lax.fori_loop(0, R, row, 0)
            for r in range(R):
                pltpu.make_async_copy(
                    rows_v.at[pl.ds(r * F, F)], out_hbm.at[base], osem
                ).wait()
            return carry

        lax.fori_loop(0, n_ch, chunk, 0)

    return lookup_kernel


def kernel(x, weight):
    B, F = x.shape
    V, D = weight.shape
    info = plsc.get_sparse_core_info()
    w3 = weight.reshape(2, V // 2, D)
    return _make_lookup(B, F, V, D, info.num_cores, info.num_subcores)(
        x, w3
    )


# speedup vs baseline: 2.0431x; 1.2611x over previous
import functools

import jax
import jax.numpy as jnp
from jax import lax
from jax.experimental import pallas as pl
from jax.experimental.pallas import tpu as pltpu
from jax.experimental.pallas import tpu_sc as plsc

_LANES = 16


def _make_lookup(B, F, V, D, NC, NS):
    NW = NC * NS
    H = V // 2
    assert B % NW == 0
    rows_w = B // NW
    R = 16
    assert rows_w % R == 0
    n_ch = rows_w // R

    mesh = plsc.VectorSubcoreMesh(core_axis_name="c", subcore_axis_name="s")

    @functools.partial(
        pl.kernel,
        mesh=mesh,
        out_type=jax.ShapeDtypeStruct((B, F, D), jnp.float32),
        scratch_types=[
            pltpu.VMEM((R, F), jnp.int32),
            pltpu.VMEM((R * F, D), jnp.float32),
            pltpu.SemaphoreType.DMA,
            pltpu.SemaphoreType.DMA,
        ],
        compiler_params=pltpu.CompilerParams(use_tc_tiling_on_sc=True),
    )
    def lookup_kernel(x_hbm, w3_hbm, out_hbm, idx_v, rows_v, gsem, osem):
        wid = lax.axis_index("s") * NC + lax.axis_index("c")
        base = wid * rows_w

        def chunk(g, carry):
            r0 = base + g * R
            pltpu.sync_copy(x_hbm.at[pl.ds(r0, R)], idx_v)

            def fire(r):
                va = idx_v[r, pl.ds(0, _LANES)]
                vb = idx_v[r, pl.ds(F - _LANES, _LANES)]
                sv = [va[j] for j in range(_LANES)]
                sv += [vb[j] for j in range(2 * _LANES - F, _LANES)]
                for f in range(F):
                    s = sv[f]
                    a = jnp.where(s >= H, 1, 0)
                    p = s - a * H
                    pltpu.async_copy(
                        w3_hbm.at[a, pl.ds(p, 1)],
                        rows_v.at[pl.ds(r * F + f, 1)],
                        gsem,
                    )

            def drain_out(r):
                for f in range(F):
                    pltpu.make_async_copy(
                        w3_hbm.at[0, pl.ds(0, 1)],
                        rows_v.at[pl.ds(0, 1)],
                        gsem,
                    ).wait()
                pltpu.async_copy(
                    rows_v.at[pl.ds(r * F, F)], out_hbm.at[r0 + r], osem
                )

            def row(r, c2):
                fire(r)

                @pl.when(r > 0)
                def _():
                    drain_out(r - 1)

                return c2

            lax.fori_loop(0, R, row, 0)
            drain_out(R - 1)
            for r in range(R):
                pltpu.make_async_copy(
                    rows_v.at[pl.ds(r * F, F)], out_hbm.at[base], osem
                ).wait()
            return carry

        lax.fori_loop(0, n_ch, chunk, 0)

    return lookup_kernel


def kernel(x, weight):
    B, F = x.shape
    V, D = weight.shape
    info = plsc.get_sparse_core_info()
    w3 = weight.reshape(2, V // 2, D)
    return _make_lookup(B, F, V, D, info.num_cores, info.num_subcores)(
        x, w3
    )


# speedup vs baseline: 2.2488x; 1.1007x over previous
import functools

import jax
import jax.numpy as jnp
from jax import lax
from jax.experimental import pallas as pl
from jax.experimental.pallas import tpu as pltpu
from jax.experimental.pallas import tpu_sc as plsc

_LANES = 16


def _make_lookup(B, F, V, D, NC, NS):
    NW = NC * NS
    H = V // 2
    assert B % NW == 0
    rows_w = B // NW
    R = 16
    assert rows_w % R == 0
    n_ch = rows_w // R

    mesh = plsc.VectorSubcoreMesh(core_axis_name="c", subcore_axis_name="s")

    @functools.partial(
        pl.kernel,
        mesh=mesh,
        out_type=jax.ShapeDtypeStruct((B, F, D), jnp.float32),
        scratch_types=[
            pltpu.VMEM((R, F), jnp.int32),
            pltpu.VMEM((R * F, D), jnp.float32),
            pltpu.SemaphoreType.DMA,
            pltpu.SemaphoreType.DMA,
        ],
        compiler_params=pltpu.CompilerParams(use_tc_tiling_on_sc=True),
    )
    def lookup_kernel(x_hbm, w3_hbm, out_hbm, idx_v, rows_v, gsem, osem):
        wid = lax.axis_index("s") * NC + lax.axis_index("c")
        base = wid * rows_w

        def chunk(g, carry):
            r0 = base + g * R
            pltpu.sync_copy(x_hbm.at[pl.ds(r0, R)], idx_v)

            def fire(r):
                va = idx_v[r, pl.ds(0, _LANES)]
                vb = idx_v[r, pl.ds(F - _LANES, _LANES)]
                sv = [va[j] for j in range(_LANES)]
                sv += [vb[j] for j in range(2 * _LANES - F, _LANES)]
                for f in range(F):
                    s = sv[f]
                    a = jnp.where(s >= H, 1, 0)
                    p = s - a * H
                    pltpu.async_copy(
                        w3_hbm.at[a, pl.ds(p, 1)],
                        rows_v.at[pl.ds(r * F + f, 1)],
                        gsem,
                    )

            def drain_out(r):
                for f in range(F):
                    pltpu.make_async_copy(
                        w3_hbm.at[0, pl.ds(0, 1)],
                        rows_v.at[pl.ds(0, 1)],
                        gsem,
                    ).wait()
                pltpu.async_copy(
                    rows_v.at[pl.ds(r * F, F)], out_hbm.at[r0 + r], osem
                )

            def row(r, c2):
                fire(r)

                @pl.when(r > 1)
                def _():
                    drain_out(r - 2)

                return c2

            lax.fori_loop(0, R, row, 0)
            drain_out(R - 2)
            drain_out(R - 1)
            for r in range(R):
                pltpu.make_async_copy(
                    rows_v.at[pl.ds(r * F, F)], out_hbm.at[base], osem
                ).wait()
            return carry

        lax.fori_loop(0, n_ch, chunk, 0)

    return lookup_kernel


def kernel(x, weight):
    B, F = x.shape
    V, D = weight.shape
    info = plsc.get_sparse_core_info()
    w3 = weight.reshape(2, V // 2, D)
    return _make_lookup(B, F, V, D, info.num_cores, info.num_subcores)(
        x, w3
    )


# speedup vs baseline: 2.3592x; 1.0491x over previous
import functools

import jax
import jax.numpy as jnp
from jax import lax
from jax.experimental import pallas as pl
from jax.experimental.pallas import tpu as pltpu
from jax.experimental.pallas import tpu_sc as plsc

_LANES = 16


def _make_lookup(B, F, V, D, NC, NS):
    NW = NC * NS
    H = V // 2
    assert B % NW == 0
    rows_w = B // NW
    R = 16
    assert rows_w % R == 0
    n_ch = rows_w // R

    mesh = plsc.VectorSubcoreMesh(core_axis_name="c", subcore_axis_name="s")

    @functools.partial(
        pl.kernel,
        mesh=mesh,
        out_type=jax.ShapeDtypeStruct((B, F, D), jnp.float32),
        scratch_types=[
            pltpu.VMEM((R, F), jnp.int32),
            pltpu.VMEM((R * F, D), jnp.float32),
            pltpu.SemaphoreType.DMA,
            pltpu.SemaphoreType.DMA,
        ],
        compiler_params=pltpu.CompilerParams(use_tc_tiling_on_sc=True),
    )
    def lookup_kernel(x_hbm, w3_hbm, out_hbm, idx_v, rows_v, gsem, osem):
        wid = lax.axis_index("s") * NC + lax.axis_index("c")
        base = wid * rows_w

        def chunk(g, carry):
            r0 = base + g * R
            pltpu.sync_copy(x_hbm.at[pl.ds(r0, R)], idx_v)

            @pl.when(g > 0)
            def _():
                for r in range(R):
                    pltpu.make_async_copy(
                        rows_v.at[pl.ds(r * F, F)], out_hbm.at[base], osem
                    ).wait()

            def fire(r):
                va = idx_v[r, pl.ds(0, _LANES)]
                vb = idx_v[r, pl.ds(F - _LANES, _LANES)]
                sv = [va[j] for j in range(_LANES)]
                sv += [vb[j] for j in range(2 * _LANES - F, _LANES)]
                for f in range(F):
                    s = sv[f]
                    a = jnp.where(s >= H, 1, 0)
                    p = s - a * H
                    pltpu.async_copy(
                        w3_hbm.at[a, pl.ds(p, 1)],
                        rows_v.at[pl.ds(r * F + f, 1)],
                        gsem,
                    )

            def drain_out(r):
                for f in range(F):
                    pltpu.make_async_copy(
                        w3_hbm.at[0, pl.ds(0, 1)],
                        rows_v.at[pl.ds(0, 1)],
                        gsem,
                    ).wait()
                pltpu.async_copy(
                    rows_v.at[pl.ds(r * F, F)], out_hbm.at[r0 + r], osem
                )

            def row(r, c2):
                fire(r)

                @pl.when(r > 2)
                def _():
                    drain_out(r - 3)

                return c2

            lax.fori_loop(0, R, row, 0)
            drain_out(R - 3)
            drain_out(R - 2)
            drain_out(R - 1)
            return carry

        lax.fori_loop(0, n_ch, chunk, 0)
        for r in range(R):
            pltpu.make_async_copy(
                rows_v.at[pl.ds(r * F, F)], out_hbm.at[base], osem
            ).wait()

    return lookup_kernel


def kernel(x, weight):
    B, F = x.shape
    V, D = weight.shape
    info = plsc.get_sparse_core_info()
    w3 = weight.reshape(2, V // 2, D)
    return _make_lookup(B, F, V, D, info.num_cores, info.num_subcores)(
        x, w3
    )
